# Initial kernel scaffold; baseline (speedup 1.0000x reference)
import jax
import jax.numpy as jnp
from jax.experimental import pallas as pl


def kernel(x_user, x_item, edge_index_user_item, edge_index_item_user, W_ui, b_ui, W_iu, b_iu, ln_g_user, ln_b_user, ln_g_item, ln_b_item):
    raise NotImplementedError("write your pallas kernel here")


# speedup vs baseline: 3.1287x; 3.1287x over previous
import functools

import jax
import jax.numpy as jnp
from jax import lax
from jax.experimental import pallas as pl
from jax.experimental.pallas import tpu as pltpu
from jax.experimental.pallas import tpu_sc as plsc

D = 128
L = 16
NC = 2
NS = 16
NW = NC * NS

GC = 128
SC_CHUNK = 80


def _gather_mul_call(x_user, x_item, src_ui, dst_ui, src_iu, dst_iu):
    E = src_ui.shape[0]
    per_w = E // NW
    n_chunks = -(-per_w // GC)
    last_base = per_w - GC

    mesh = plsc.VectorSubcoreMesh(core_axis_name="c", subcore_axis_name="s")

    @functools.partial(
        pl.kernel,
        mesh=mesh,
        out_type=(jax.ShapeDtypeStruct((E, D), jnp.float32),
                  jax.ShapeDtypeStruct((E, D), jnp.float32)),
        scratch_types=[
            pltpu.VMEM((GC,), jnp.int32),
            pltpu.VMEM((GC,), jnp.int32),
            pltpu.VMEM((GC, D), jnp.float32),
            pltpu.VMEM((GC, D), jnp.float32),
            pltpu.VMEM((GC, D), jnp.float32),
            pltpu.SemaphoreType.DMA,
            pltpu.SemaphoreType.DMA,
        ],
    )
    def gather_mul(xu_hbm, xi_hbm, sui_hbm, dui_hbm, siu_hbm, diu_hbm,
                   p_ui_hbm, p_iu_hbm, si_v, di_v, xj_v, xd_v, p_v, sem1, sem2):
        wid = lax.axis_index("s") * NC + lax.axis_index("c")
        w_base = wid * per_w

        def one_type(xs_hbm, xd_hbm, s_hbm, d_hbm, p_hbm):
            def chunk_body(i, _):
                base = pl.multiple_of(w_base + jnp.minimum(i * GC, last_base), 8)
                pltpu.sync_copy(s_hbm.at[pl.ds(base, GC)], si_v)
                pltpu.sync_copy(d_hbm.at[pl.ds(base, GC)], di_v)
                cp1 = pltpu.async_copy(xs_hbm.at[si_v], xj_v, sem1)
                cp2 = pltpu.async_copy(xd_hbm.at[di_v], xd_v, sem2)
                cp1.wait()
                cp2.wait()

                def row_body(r, _):
                    for j in range(D // L):
                        sl = pl.ds(j * L, L)
                        p_v[r, sl] = xj_v[r, sl] * xd_v[r, sl]
                    return 0

                lax.fori_loop(0, GC, row_body, 0)
                pltpu.sync_copy(p_v, p_hbm.at[pl.ds(base, GC)])
                return 0

            lax.fori_loop(0, n_chunks, chunk_body, 0)

        one_type(xu_hbm, xi_hbm, sui_hbm, dui_hbm, p_ui_hbm)
        one_type(xi_hbm, xu_hbm, siu_hbm, diu_hbm, p_iu_hbm)

    return gather_mul(x_user, x_item, src_ui, dst_ui, src_iu, dst_iu)


def _linear_lrelu_call(p_ui, p_iu, W_s, b_s):
    E = p_ui.shape[0]
    BLK = 2000

    def body(p1_ref, p2_ref, W_ref, b_ref, o1_ref, o2_ref):
        W = W_ref[...]
        b = b_ref[...]
        dn = (((1,), (1,)), ((), ()))
        z1 = lax.dot_general(p1_ref[...], W[0], dn,
                             preferred_element_type=jnp.float32) + b[0]
        z2 = lax.dot_general(p2_ref[...], W[1], dn,
                             preferred_element_type=jnp.float32) + b[1]
        o1_ref[...] = jnp.where(z1 >= 0, z1, 0.01 * z1)
        o2_ref[...] = jnp.where(z2 >= 0, z2, 0.01 * z2)

    grid = (E // BLK,)
    return pl.pallas_call(
        body,
        grid=grid,
        in_specs=[
            pl.BlockSpec((BLK, D), lambda i: (i, 0)),
            pl.BlockSpec((BLK, D), lambda i: (i, 0)),
            pl.BlockSpec((2, D, D), lambda i: (0, 0, 0)),
            pl.BlockSpec((2, D), lambda i: (0, 0)),
        ],
        out_specs=[
            pl.BlockSpec((BLK, D), lambda i: (i, 0)),
            pl.BlockSpec((BLK, D), lambda i: (i, 0)),
        ],
        out_shape=(jax.ShapeDtypeStruct((E, D), jnp.float32),
                   jax.ShapeDtypeStruct((E, D), jnp.float32)),
    )(p_ui, p_iu, W_s, b_s)


def _scatter_call(m_ui, dst_ui, m_iu, dst_iu, zeros_nd, n_item, n_user):
    E = m_ui.shape[0]
    per_t = E // NS
    n_chunks = per_t // SC_CHUNK
    n_rows = n_item
    stripe = 624
    stripe_last = n_rows - (NS - 1) * stripe

    mesh = plsc.VectorSubcoreMesh(core_axis_name="c", subcore_axis_name="s")

    @functools.partial(
        pl.kernel,
        mesh=mesh,
        out_type=(jax.ShapeDtypeStruct((n_item, D), jnp.float32),
                  jax.ShapeDtypeStruct((n_user, D), jnp.float32)),
        scratch_types=[
            pltpu.VMEM((SC_CHUNK,), jnp.int32),
            pltpu.VMEM((SC_CHUNK, D), jnp.float32),
            pltpu.VMEM_SHARED((10000, D), jnp.float32),
        ],
    )
    def scatter(mui_hbm, dui_hbm, miu_hbm, diu_hbm, z_hbm,
                oi_hbm, ou_hbm, di_v, mrow_v, acc_sh):
        cid = lax.axis_index("c")
        tid = lax.axis_index("s")

        @pl.when(tid < NS - 1)
        def _():
            off = pl.multiple_of(tid * stripe, 8)
            pltpu.sync_copy(z_hbm.at[pl.ds(off, stripe)],
                            acc_sh.at[pl.ds(off, stripe)])

        @pl.when(tid == NS - 1)
        def _():
            off = (NS - 1) * stripe
            pltpu.sync_copy(z_hbm.at[pl.ds(off, stripe_last)],
                            acc_sh.at[pl.ds(off, stripe_last)])

        plsc.subcore_barrier()

        def one_type(m_hbm, d_hbm):
            t_base = tid * per_t

            def chunk_body(i, _):
                base = pl.multiple_of(t_base + i * SC_CHUNK, 8)
                pltpu.sync_copy(d_hbm.at[pl.ds(base, SC_CHUNK)], di_v)
                pltpu.sync_copy(m_hbm.at[pl.ds(base, SC_CHUNK)], mrow_v)
                pltpu.sync_copy(mrow_v, acc_sh.at[di_v], add=True)
                return 0

            lax.fori_loop(0, n_chunks, chunk_body, 0)

        @pl.when(cid == 0)
        def _():
            one_type(mui_hbm, dui_hbm)

        @pl.when(cid == 1)
        def _():
            one_type(miu_hbm, diu_hbm)

        plsc.subcore_barrier()

        def dump(o_hbm):
            @pl.when(tid < NS - 1)
            def _():
                off = pl.multiple_of(tid * stripe, 8)
                pltpu.sync_copy(acc_sh.at[pl.ds(off, stripe)],
                                o_hbm.at[pl.ds(off, stripe)])

            @pl.when(tid == NS - 1)
            def _():
                off = (NS - 1) * stripe
                pltpu.sync_copy(acc_sh.at[pl.ds(off, stripe_last)],
                                o_hbm.at[pl.ds(off, stripe_last)])

        @pl.when(cid == 0)
        def _():
            dump(oi_hbm)

        @pl.when(cid == 1)
        def _():
            dump(ou_hbm)

    return scatter(m_ui, dst_ui, m_iu, dst_iu, zeros_nd)


def _ln_relu_call(oi_raw, ou_raw, ln_g_item, ln_b_item, ln_g_user, ln_b_user):
    n = oi_raw.shape[0]
    BLKN = 2000

    def body(x1_ref, x2_ref, g1_ref, b1_ref, g2_ref, b2_ref, o1_ref, o2_ref):
        for x_ref, g_ref, b_ref, o_ref in (
            (x1_ref, g1_ref, b1_ref, o1_ref),
            (x2_ref, g2_ref, b2_ref, o2_ref),
        ):
            x = x_ref[...]
            mu = jnp.mean(x, axis=-1, keepdims=True)
            var = jnp.mean((x - mu) ** 2, axis=-1, keepdims=True)
            y = (x - mu) / jnp.sqrt(var + 1e-5) * g_ref[...] + b_ref[...]
            o_ref[...] = jnp.maximum(y, 0.0)

    grid = (n // BLKN,)
    vec_spec = pl.BlockSpec((1, D), lambda i: (0, 0))
    return pl.pallas_call(
        body,
        grid=grid,
        in_specs=[
            pl.BlockSpec((BLKN, D), lambda i: (i, 0)),
            pl.BlockSpec((BLKN, D), lambda i: (i, 0)),
            vec_spec, vec_spec, vec_spec, vec_spec,
        ],
        out_specs=[
            pl.BlockSpec((BLKN, D), lambda i: (i, 0)),
            pl.BlockSpec((BLKN, D), lambda i: (i, 0)),
        ],
        out_shape=(jax.ShapeDtypeStruct((n, D), jnp.float32),
                   jax.ShapeDtypeStruct((n, D), jnp.float32)),
    )(oi_raw, ou_raw, ln_g_item.reshape(1, D), ln_b_item.reshape(1, D),
      ln_g_user.reshape(1, D), ln_b_user.reshape(1, D))


def kernel(x_user, x_item, edge_index_user_item, edge_index_item_user,
           W_ui, b_ui, W_iu, b_iu,
           ln_g_user, ln_b_user, ln_g_item, ln_b_item):
    n_user = x_user.shape[0]
    n_item = x_item.shape[0]

    src_ui = edge_index_user_item[0].astype(jnp.int32)
    dst_ui = edge_index_user_item[1].astype(jnp.int32)
    src_iu = edge_index_item_user[0].astype(jnp.int32)
    dst_iu = edge_index_item_user[1].astype(jnp.int32)

    p_ui, p_iu = _gather_mul_call(x_user, x_item, src_ui, dst_ui, src_iu, dst_iu)

    W_s = jnp.stack([W_ui, W_iu])
    b_s = jnp.stack([b_ui, b_iu])
    m_ui, m_iu = _linear_lrelu_call(p_ui, p_iu, W_s, b_s)

    zeros_nd = jnp.zeros((n_item, D), jnp.float32)
    oi_raw, ou_raw = _scatter_call(m_ui, dst_ui, m_iu, dst_iu, zeros_nd,
                                   n_item, n_user)

    out_item, out_user = _ln_relu_call(oi_raw, ou_raw,
                                       ln_g_item, ln_b_item,
                                       ln_g_user, ln_b_user)
    return (out_user, out_item)


# speedup vs baseline: 5.0276x; 1.6069x over previous
import functools

import jax
import jax.numpy as jnp
from jax import lax
from jax.experimental import pallas as pl
from jax.experimental.pallas import tpu as pltpu
from jax.experimental.pallas import tpu_sc as plsc

D = 128
L = 16
NC = 2
NS = 16
NW = NC * NS

GC = 128
SC_CHUNK = 80


def _gather_mul_call(x_user, x_item, src_ui, dst_ui, src_iu, dst_iu):
    E = src_ui.shape[0]
    per_w = E // NW
    n_chunks = -(-per_w // GC)
    last_base = per_w - GC
    assert n_chunks % 2 == 0

    mesh = plsc.VectorSubcoreMesh(core_axis_name="c", subcore_axis_name="s")

    @functools.partial(
        pl.kernel,
        mesh=mesh,
        out_type=(jax.ShapeDtypeStruct((E, D), jnp.float32),
                  jax.ShapeDtypeStruct((E, D), jnp.float32)),
        scratch_types=[
            pltpu.VMEM((per_w,), jnp.int32),
            pltpu.VMEM((per_w,), jnp.int32),
            pltpu.VMEM((GC, D), jnp.float32),
            pltpu.VMEM((GC, D), jnp.float32),
            pltpu.VMEM((GC, D), jnp.float32),
            pltpu.VMEM((GC, D), jnp.float32),
            pltpu.VMEM((GC, D), jnp.float32),
            pltpu.VMEM((GC, D), jnp.float32),
            pltpu.SemaphoreType.DMA,
            pltpu.SemaphoreType.DMA,
            pltpu.SemaphoreType.DMA,
            pltpu.SemaphoreType.DMA,
        ],
    )
    def gather_mul(xu_hbm, xi_hbm, sui_hbm, dui_hbm, siu_hbm, diu_hbm,
                   p_ui_hbm, p_iu_hbm,
                   si_all, di_all, xj0, xj1, xd0, xd1, p0, p1,
                   gs0, gs1, ss0, ss1):
        xj = (xj0, xj1)
        xd = (xd0, xd1)
        pv = (p0, p1)
        gsem = (gs0, gs1)
        ssem = (ss0, ss1)
        wid = lax.axis_index("s") * NC + lax.axis_index("c")
        w_base = wid * per_w

        def one_type(xs_hbm, xdst_hbm, s_hbm, d_hbm, p_hbm, first):
            pltpu.sync_copy(s_hbm.at[pl.ds(w_base, per_w)], si_all)
            pltpu.sync_copy(d_hbm.at[pl.ds(w_base, per_w)], di_all)

            def off(c):
                return pl.multiple_of(jnp.minimum(c * GC, last_base), 8)

            def start_gathers(c, s):
                o = off(c)
                pltpu.async_copy(xs_hbm.at[si_all.at[pl.ds(o, GC)]], xj[s], gsem[s])
                pltpu.async_copy(xdst_hbm.at[di_all.at[pl.ds(o, GC)]], xd[s], gsem[s])

            def wait_gathers(s):
                z = pl.ds(0, GC)
                pltpu.make_async_copy(xs_hbm.at[si_all.at[z]], xj[s], gsem[s]).wait()
                pltpu.make_async_copy(xdst_hbm.at[di_all.at[z]], xd[s], gsem[s]).wait()

            def wait_store(s):
                pltpu.make_async_copy(pv[s], p_hbm.at[pl.ds(0, GC)], ssem[s]).wait()

            start_gathers(0, 0)

            def pair_body(i2, _):
                for s in (0, 1):
                    c = 2 * i2 + s

                    @pl.when(c + 1 < n_chunks)
                    def _():
                        start_gathers(c + 1, 1 - s)

                    wait_gathers(s)
                    if first:
                        @pl.when(c >= 2)
                        def _():
                            wait_store(s)
                    else:
                        wait_store(s)

                    @plsc.parallel_loop(0, GC, unroll=8)
                    def _(r):
                        for j in range(D // L):
                            sl = pl.ds(j * L, L)
                            pv[s][r, sl] = xj[s][r, sl] * xd[s][r, sl]

                    base = pl.multiple_of(w_base + off(c), 8)
                    pltpu.async_copy(pv[s], p_hbm.at[pl.ds(base, GC)], ssem[s])
                return 0

            lax.fori_loop(0, n_chunks // 2, pair_body, 0)

        one_type(xu_hbm, xi_hbm, sui_hbm, dui_hbm, p_ui_hbm, True)
        one_type(xi_hbm, xu_hbm, siu_hbm, diu_hbm, p_iu_hbm, False)
        pltpu.make_async_copy(pv[0], p_iu_hbm.at[pl.ds(0, GC)], ssem[0]).wait()
        pltpu.make_async_copy(pv[1], p_iu_hbm.at[pl.ds(0, GC)], ssem[1]).wait()

    return gather_mul(x_user, x_item, src_ui, dst_ui, src_iu, dst_iu)


def _linear_lrelu_call(p_ui, p_iu, W_s, b_s):
    E = p_ui.shape[0]
    BLK = 2000

    def body(p1_ref, p2_ref, W_ref, b_ref, o1_ref, o2_ref):
        W = W_ref[...]
        b = b_ref[...]
        dn = (((1,), (1,)), ((), ()))
        z1 = lax.dot_general(p1_ref[...], W[0], dn,
                             preferred_element_type=jnp.float32) + b[0]
        z2 = lax.dot_general(p2_ref[...], W[1], dn,
                             preferred_element_type=jnp.float32) + b[1]
        o1_ref[...] = jnp.where(z1 >= 0, z1, 0.01 * z1)
        o2_ref[...] = jnp.where(z2 >= 0, z2, 0.01 * z2)

    grid = (E // BLK,)
    return pl.pallas_call(
        body,
        grid=grid,
        in_specs=[
            pl.BlockSpec((BLK, D), lambda i: (i, 0)),
            pl.BlockSpec((BLK, D), lambda i: (i, 0)),
            pl.BlockSpec((2, D, D), lambda i: (0, 0, 0)),
            pl.BlockSpec((2, D), lambda i: (0, 0)),
        ],
        out_specs=[
            pl.BlockSpec((BLK, D), lambda i: (i, 0)),
            pl.BlockSpec((BLK, D), lambda i: (i, 0)),
        ],
        out_shape=(jax.ShapeDtypeStruct((E, D), jnp.float32),
                   jax.ShapeDtypeStruct((E, D), jnp.float32)),
    )(p_ui, p_iu, W_s, b_s)


def _scatter_call(m_ui, dst2_ui, m_iu, dst2_iu, zeros_nd, n_item, n_user):
    E = m_ui.shape[0]
    per_t = E // NS
    n_chunks = per_t // SC_CHUNK
    n_rows = n_item
    stripe = 624
    stripe_last = n_rows - (NS - 1) * stripe

    mesh = plsc.VectorSubcoreMesh(core_axis_name="c", subcore_axis_name="s")

    @functools.partial(
        pl.kernel,
        mesh=mesh,
        out_type=(jax.ShapeDtypeStruct((n_item, D), jnp.float32),
                  jax.ShapeDtypeStruct((n_user, D), jnp.float32)),
        scratch_types=[
            pltpu.VMEM((per_t // SC_CHUNK, SC_CHUNK), jnp.int32),
            pltpu.VMEM((SC_CHUNK, D), jnp.float32),
            pltpu.VMEM((SC_CHUNK, D), jnp.float32),
            pltpu.VMEM_SHARED((10000, D), jnp.float32),
            pltpu.SemaphoreType.DMA,
            pltpu.SemaphoreType.DMA,
            pltpu.SemaphoreType.DMA,
            pltpu.SemaphoreType.DMA,
        ],
    )
    def scatter(mui_hbm, dui_hbm, miu_hbm, diu_hbm, z_hbm,
                oi_hbm, ou_hbm, d_all, mrow0, mrow1, acc_sh,
                ls0, ls1, cs0, cs1):
        mrow = (mrow0, mrow1)
        lsem = (ls0, ls1)
        csem = (cs0, cs1)
        cid = lax.axis_index("c")
        tid = lax.axis_index("s")

        @pl.when(tid < NS - 1)
        def _():
            off = pl.multiple_of(tid * stripe, 8)
            pltpu.sync_copy(z_hbm.at[pl.ds(off, stripe)],
                            acc_sh.at[pl.ds(off, stripe)])

        @pl.when(tid == NS - 1)
        def _():
            off = (NS - 1) * stripe
            pltpu.sync_copy(z_hbm.at[pl.ds(off, stripe_last)],
                            acc_sh.at[pl.ds(off, stripe_last)])

        plsc.subcore_barrier()

        def one_type(m_hbm, d_hbm):
            t_base = tid * per_t
            pltpu.sync_copy(d_hbm.at[tid], d_all)

            def start_load(c, s):
                base = pl.multiple_of(t_base + c * SC_CHUNK, 8)
                pltpu.async_copy(m_hbm.at[pl.ds(base, SC_CHUNK)], mrow[s], lsem[s])

            def wait_load(s):
                pltpu.make_async_copy(m_hbm.at[pl.ds(t_base, SC_CHUNK)],
                                      mrow[s], lsem[s]).wait()

            def wait_scat(s):
                pltpu.make_async_copy(mrow[s], acc_sh.at[d_all.at[0]],
                                      csem[s]).wait()

            def do_chunk(c, s, may_be_first):
                wait_load(s)
                pltpu.async_copy(mrow[s], acc_sh.at[d_all.at[c]], csem[s],
                                 add=True)
                if may_be_first:
                    @pl.when(c >= 1)
                    def _():
                        wait_scat(1 - s)
                else:
                    wait_scat(1 - s)

                @pl.when(c + 1 < n_chunks)
                def _():
                    start_load(c + 1, 1 - s)

            start_load(0, 0)

            def pair_body(i2, _):
                c = 2 * i2
                do_chunk(c, 0, True)
                do_chunk(c + 1, 1, False)
                return 0

            lax.fori_loop(0, n_chunks // 2, pair_body, 0)
            if n_chunks % 2 == 1:
                do_chunk(n_chunks - 1, 0, False)
            wait_scat((n_chunks - 1) % 2)

        @pl.when(cid == 0)
        def _():
            one_type(mui_hbm, dui_hbm)

        @pl.when(cid == 1)
        def _():
            one_type(miu_hbm, diu_hbm)

        plsc.subcore_barrier()

        def dump(o_hbm):
            @pl.when(tid < NS - 1)
            def _():
                off = pl.multiple_of(tid * stripe, 8)
                pltpu.sync_copy(acc_sh.at[pl.ds(off, stripe)],
                                o_hbm.at[pl.ds(off, stripe)])

            @pl.when(tid == NS - 1)
            def _():
                off = (NS - 1) * stripe
                pltpu.sync_copy(acc_sh.at[pl.ds(off, stripe_last)],
                                o_hbm.at[pl.ds(off, stripe_last)])

        @pl.when(cid == 0)
        def _():
            dump(oi_hbm)

        @pl.when(cid == 1)
        def _():
            dump(ou_hbm)

    return scatter(m_ui, dst2_ui, m_iu, dst2_iu, zeros_nd)


def _ln_relu_call(oi_raw, ou_raw, ln_g_item, ln_b_item, ln_g_user, ln_b_user):
    n = oi_raw.shape[0]
    BLKN = 2000

    def body(x1_ref, x2_ref, g1_ref, b1_ref, g2_ref, b2_ref, o1_ref, o2_ref):
        for x_ref, g_ref, b_ref, o_ref in (
            (x1_ref, g1_ref, b1_ref, o1_ref),
            (x2_ref, g2_ref, b2_ref, o2_ref),
        ):
            x = x_ref[...]
            mu = jnp.mean(x, axis=-1, keepdims=True)
            var = jnp.mean((x - mu) ** 2, axis=-1, keepdims=True)
            y = (x - mu) / jnp.sqrt(var + 1e-5) * g_ref[...] + b_ref[...]
            o_ref[...] = jnp.maximum(y, 0.0)

    grid = (n // BLKN,)
    vec_spec = pl.BlockSpec((1, D), lambda i: (0, 0))
    return pl.pallas_call(
        body,
        grid=grid,
        in_specs=[
            pl.BlockSpec((BLKN, D), lambda i: (i, 0)),
            pl.BlockSpec((BLKN, D), lambda i: (i, 0)),
            vec_spec, vec_spec, vec_spec, vec_spec,
        ],
        out_specs=[
            pl.BlockSpec((BLKN, D), lambda i: (i, 0)),
            pl.BlockSpec((BLKN, D), lambda i: (i, 0)),
        ],
        out_shape=(jax.ShapeDtypeStruct((n, D), jnp.float32),
                   jax.ShapeDtypeStruct((n, D), jnp.float32)),
    )(oi_raw, ou_raw, ln_g_item.reshape(1, D), ln_b_item.reshape(1, D),
      ln_g_user.reshape(1, D), ln_b_user.reshape(1, D))


def kernel(x_user, x_item, edge_index_user_item, edge_index_item_user,
           W_ui, b_ui, W_iu, b_iu,
           ln_g_user, ln_b_user, ln_g_item, ln_b_item):
    n_user = x_user.shape[0]
    n_item = x_item.shape[0]

    src_ui = edge_index_user_item[0].astype(jnp.int32)
    dst_ui = edge_index_user_item[1].astype(jnp.int32)
    src_iu = edge_index_item_user[0].astype(jnp.int32)
    dst_iu = edge_index_item_user[1].astype(jnp.int32)

    p_ui, p_iu = _gather_mul_call(x_user, x_item, src_ui, dst_ui, src_iu, dst_iu)

    W_s = jnp.stack([W_ui, W_iu])
    b_s = jnp.stack([b_ui, b_iu])
    m_ui, m_iu = _linear_lrelu_call(p_ui, p_iu, W_s, b_s)

    zeros_nd = jnp.zeros((n_item, D), jnp.float32)
    dst2_ui = dst_ui.reshape(NS, -1, SC_CHUNK)
    dst2_iu = dst_iu.reshape(NS, -1, SC_CHUNK)
    oi_raw, ou_raw = _scatter_call(m_ui, dst2_ui, m_iu, dst2_iu, zeros_nd,
                                   n_item, n_user)

    out_item, out_user = _ln_relu_call(oi_raw, ou_raw,
                                       ln_g_item, ln_b_item,
                                       ln_g_user, ln_b_user)
    return (out_user, out_item)
